# baseline (speedup 1.0000x reference)
import jax
import jax.numpy as jnp
from jax import lax
from jax.experimental import pallas as pl
from jax.experimental.pallas import tpu as pltpu
from jax.experimental.pallas import tpu_sc as plsc

N = 10000
E = 320000
D = 128
BN_EPS = 1e-5

NC = 2
NS = 16
CHUNK = 40
K = 250
ROWS_PER_SUB = 632
N_PAD = NS * ROWS_PER_SUB
NBUF = 2
IBLK = 10
NBLK = K // IBLK


def _sc_agg_kernel(x_hbm, row_hbm, col_hbm, z_hbm, out_hbm,
                   ir0, ir1, ic0, ic1, buf0, buf1, zbuf,
                   gsem0, gsem1, isem0, isem1, agg_sh):
    c = lax.axis_index("c")
    s = lax.axis_index("s")
    bufs = (buf0, buf1)
    gsems = (gsem0, gsem1)
    irows = (ir0, ir1)
    icols = (ic0, ic1)
    isems = (isem0, isem1)

    pltpu.sync_copy(row_hbm.at[c, s, 0], ir0)
    pltpu.sync_copy(col_hbm.at[c, s, 0], ic0)
    pltpu.async_copy(row_hbm.at[c, s, 1], ir1, isem1)
    pltpu.async_copy(col_hbm.at[c, s, 1], ic1, isem1)
    for b in range(NBUF):
        pltpu.async_copy(x_hbm.at[ic0.at[b]], bufs[b], gsems[b])

    pltpu.sync_copy(z_hbm, zbuf)

    r0 = s * ROWS_PER_SUB
    for off in range(0, 576, 64):
        pltpu.sync_copy(zbuf, agg_sh.at[pl.ds(r0 + off, 64)])
    pltpu.sync_copy(zbuf.at[pl.ds(0, 56)], agg_sh.at[pl.ds(r0 + 576, 56)])
    plsc.subcore_barrier()

    for ib in range(NBLK):
        ic, ir = icols[ib % 2], irows[ib % 2]

        @pl.loop(0, IBLK - NBUF, step=NBUF)
        def _(j0):
            for b in range(NBUF):
                j = j0 + b
                pltpu.make_async_copy(x_hbm.at[ic.at[j]], bufs[b],
                                      gsems[b]).wait()
                pltpu.sync_copy(bufs[b], agg_sh.at[ir.at[j]], add=True)
                pltpu.async_copy(x_hbm.at[ic.at[j + NBUF]], bufs[b], gsems[b])

        nxt = (ib + 1) % 2
        if ib + 1 < NBLK:
            pltpu.make_async_copy(row_hbm.at[c, s, 0],
                                  irows[nxt], isems[nxt]).wait()
            pltpu.make_async_copy(col_hbm.at[c, s, 0],
                                  icols[nxt], isems[nxt]).wait()
        for b in range(NBUF):
            j = IBLK - NBUF + b
            pltpu.make_async_copy(x_hbm.at[ic.at[j]], bufs[b], gsems[b]).wait()
            pltpu.sync_copy(bufs[b], agg_sh.at[ir.at[j]], add=True)
            if ib + 1 < NBLK:
                pltpu.async_copy(x_hbm.at[icols[nxt].at[b]], bufs[b], gsems[b])
        if ib + 2 < NBLK:
            pltpu.async_copy(row_hbm.at[c, s, ib + 2], ir, isems[ib % 2])
            pltpu.async_copy(col_hbm.at[c, s, ib + 2], ic, isems[ib % 2])

    pltpu.sync_copy(zbuf.at[pl.ds(0, CHUNK)],
                    agg_sh.at[irows[(NBLK - 1) % 2].at[IBLK - 1]], add=True)
    plsc.subcore_barrier()

    pltpu.sync_copy(agg_sh.at[pl.ds(r0, ROWS_PER_SUB)],
                    out_hbm.at[c, pl.ds(r0, ROWS_PER_SUB)])


def _sc_agg(x, row4, col4):
    mesh = plsc.VectorSubcoreMesh(core_axis_name="c", subcore_axis_name="s")
    kern = pl.kernel(
        _sc_agg_kernel,
        out_type=jax.ShapeDtypeStruct((NC, N_PAD, D), jnp.float32),
        mesh=mesh,
        scratch_types=[
            pltpu.VMEM((IBLK, CHUNK), jnp.int32),
            pltpu.VMEM((IBLK, CHUNK), jnp.int32),
            pltpu.VMEM((IBLK, CHUNK), jnp.int32),
            pltpu.VMEM((IBLK, CHUNK), jnp.int32),
            pltpu.VMEM((CHUNK, D), jnp.float32),
            pltpu.VMEM((CHUNK, D), jnp.float32),
            pltpu.VMEM((64, D), jnp.float32),
            pltpu.SemaphoreType.DMA,
            pltpu.SemaphoreType.DMA,
            pltpu.SemaphoreType.DMA,
            pltpu.SemaphoreType.DMA,
            pltpu.VMEM_SHARED((N_PAD, D), jnp.float32),
        ],
    )
    return kern(x, row4, col4, jnp.zeros((64, D), jnp.float32))


def _tc_mlp_kernel(x_ref, agg_ref, eps_ref, w1_ref, b1_ref, g1_ref, be1_ref,
                   w2_ref, b2_ref, g2_ref, be2_ref, o_ref):
    x = x_ref[...]
    agg = agg_ref[0, :N, :] + agg_ref[1, :N, :]
    out = (1.0 + eps_ref[0, 0]) * x + agg
    h = jnp.dot(out, w1_ref[...], preferred_element_type=jnp.float32)
    h = h + b1_ref[...]
    mu = jnp.mean(h, axis=0, keepdims=True)
    var = jnp.mean((h - mu) * (h - mu), axis=0, keepdims=True)
    h = (h - mu) * lax.rsqrt(var + BN_EPS) * g1_ref[...] + be1_ref[...]
    h = jnp.maximum(h, 0.0)
    h2 = jnp.dot(h, w2_ref[...], preferred_element_type=jnp.float32)
    h2 = h2 + b2_ref[...]
    mu2 = jnp.mean(h2, axis=0, keepdims=True)
    var2 = jnp.mean((h2 - mu2) * (h2 - mu2), axis=0, keepdims=True)
    o_ref[...] = (h2 - mu2) * lax.rsqrt(var2 + BN_EPS) * g2_ref[...] + be2_ref[...]


def _tc_mlp(x, agg, eps, w1t, b1, g1, be1, w2t, b2, g2, be2):
    return pl.pallas_call(
        _tc_mlp_kernel,
        out_shape=jax.ShapeDtypeStruct((N, D), jnp.float32),
    )(x, agg, eps, w1t, b1, g1, be1, w2t, b2, g2, be2)


@jax.jit
def kernel(x, edge_index, eps, W1, b1, g1, be1, W2, b2, g2, be2):
    row4 = edge_index[0].reshape(NC, NS, NBLK, IBLK, CHUNK)
    col4 = edge_index[1].reshape(NC, NS, NBLK, IBLK, CHUNK)
    agg = _sc_agg(x, row4, col4)

    return _tc_mlp(x, agg, eps.reshape(1, 1), W1.T, b1.reshape(1, D),
                   g1.reshape(1, D), be1.reshape(1, D), W2.T,
                   b2.reshape(1, D), g2.reshape(1, D), be2.reshape(1, D))


# speedup vs baseline: 1.4300x; 1.4300x over previous
import jax
import jax.numpy as jnp
from jax import lax
from jax.experimental import pallas as pl
from jax.experimental.pallas import tpu as pltpu
from jax.experimental.pallas import tpu_sc as plsc

N = 10000
E = 320000
D = 128
BN_EPS = 1e-5

NC = 2
NS = 16
CHUNK = 40
K = 250
ROWS_PER_SUB = 632
N_PAD = NS * ROWS_PER_SUB
NBUF = 5
IBLK = 10
NBLK = K // IBLK


def _sc_agg_kernel(x_hbm, row_hbm, col_hbm, z_hbm, out_hbm,
                   ir0, ir1, ic0, ic1, buf0, buf1, buf2, buf3, buf4, zbuf,
                   gsem0, gsem1, gsem2, gsem3, gsem4, isem0, isem1, agg_sh):
    c = lax.axis_index("c")
    s = lax.axis_index("s")
    bufs = (buf0, buf1, buf2, buf3, buf4)
    gsems = (gsem0, gsem1, gsem2, gsem3, gsem4)
    irows = (ir0, ir1)
    icols = (ic0, ic1)
    isems = (isem0, isem1)

    pltpu.sync_copy(row_hbm.at[c, s, 0], ir0)
    pltpu.sync_copy(col_hbm.at[c, s, 0], ic0)
    pltpu.async_copy(row_hbm.at[c, s, 1], ir1, isem1)
    pltpu.async_copy(col_hbm.at[c, s, 1], ic1, isem1)
    for b in range(NBUF):
        pltpu.async_copy(x_hbm.at[ic0.at[b]], bufs[b], gsems[b])

    pltpu.sync_copy(z_hbm, zbuf)

    r0 = s * ROWS_PER_SUB
    for off in range(0, 576, 64):
        pltpu.sync_copy(zbuf, agg_sh.at[pl.ds(r0 + off, 64)])
    pltpu.sync_copy(zbuf.at[pl.ds(0, 56)], agg_sh.at[pl.ds(r0 + 576, 56)])
    plsc.subcore_barrier()

    for ib in range(NBLK):
        ic, ir = icols[ib % 2], irows[ib % 2]

        @pl.loop(0, IBLK - NBUF, step=NBUF)
        def _(j0):
            for b in range(NBUF):
                j = j0 + b
                pltpu.make_async_copy(x_hbm.at[ic.at[j]], bufs[b],
                                      gsems[b]).wait()
                pltpu.sync_copy(bufs[b], agg_sh.at[ir.at[j]], add=True)
                pltpu.async_copy(x_hbm.at[ic.at[j + NBUF]], bufs[b], gsems[b])

        nxt = (ib + 1) % 2
        if ib + 1 < NBLK:
            pltpu.make_async_copy(row_hbm.at[c, s, 0],
                                  irows[nxt], isems[nxt]).wait()
            pltpu.make_async_copy(col_hbm.at[c, s, 0],
                                  icols[nxt], isems[nxt]).wait()
        for b in range(NBUF):
            j = IBLK - NBUF + b
            pltpu.make_async_copy(x_hbm.at[ic.at[j]], bufs[b], gsems[b]).wait()
            pltpu.sync_copy(bufs[b], agg_sh.at[ir.at[j]], add=True)
            if ib + 1 < NBLK:
                pltpu.async_copy(x_hbm.at[icols[nxt].at[b]], bufs[b], gsems[b])
        if ib + 2 < NBLK:
            pltpu.async_copy(row_hbm.at[c, s, ib + 2], ir, isems[ib % 2])
            pltpu.async_copy(col_hbm.at[c, s, ib + 2], ic, isems[ib % 2])

    pltpu.sync_copy(zbuf.at[pl.ds(0, CHUNK)],
                    agg_sh.at[irows[(NBLK - 1) % 2].at[IBLK - 1]], add=True)
    plsc.subcore_barrier()

    pltpu.sync_copy(agg_sh.at[pl.ds(r0, ROWS_PER_SUB)],
                    out_hbm.at[c, pl.ds(r0, ROWS_PER_SUB)])


def _sc_agg(x, row4, col4):
    mesh = plsc.VectorSubcoreMesh(core_axis_name="c", subcore_axis_name="s")
    kern = pl.kernel(
        _sc_agg_kernel,
        out_type=jax.ShapeDtypeStruct((NC, N_PAD, D), jnp.float32),
        mesh=mesh,
        scratch_types=[
            pltpu.VMEM((IBLK, CHUNK), jnp.int32),
            pltpu.VMEM((IBLK, CHUNK), jnp.int32),
            pltpu.VMEM((IBLK, CHUNK), jnp.int32),
            pltpu.VMEM((IBLK, CHUNK), jnp.int32),
            pltpu.VMEM((CHUNK, D), jnp.float32),
            pltpu.VMEM((CHUNK, D), jnp.float32),
            pltpu.VMEM((CHUNK, D), jnp.float32),
            pltpu.VMEM((CHUNK, D), jnp.float32),
            pltpu.VMEM((CHUNK, D), jnp.float32),
            pltpu.VMEM((64, D), jnp.float32),
            pltpu.SemaphoreType.DMA,
            pltpu.SemaphoreType.DMA,
            pltpu.SemaphoreType.DMA,
            pltpu.SemaphoreType.DMA,
            pltpu.SemaphoreType.DMA,
            pltpu.SemaphoreType.DMA,
            pltpu.SemaphoreType.DMA,
            pltpu.VMEM_SHARED((N_PAD, D), jnp.float32),
        ],
    )
    return kern(x, row4, col4, jnp.zeros((64, D), jnp.float32))


def _tc_mlp_kernel(x_ref, agg_ref, eps_ref, w1_ref, b1_ref, g1_ref, be1_ref,
                   w2_ref, b2_ref, g2_ref, be2_ref, o_ref):
    x = x_ref[...]
    agg = agg_ref[0, :N, :] + agg_ref[1, :N, :]
    out = (1.0 + eps_ref[0, 0]) * x + agg
    h = jnp.dot(out, w1_ref[...], preferred_element_type=jnp.float32)
    h = h + b1_ref[...]
    mu = jnp.mean(h, axis=0, keepdims=True)
    var = jnp.mean((h - mu) * (h - mu), axis=0, keepdims=True)
    h = (h - mu) * lax.rsqrt(var + BN_EPS) * g1_ref[...] + be1_ref[...]
    h = jnp.maximum(h, 0.0)
    h2 = jnp.dot(h, w2_ref[...], preferred_element_type=jnp.float32)
    h2 = h2 + b2_ref[...]
    mu2 = jnp.mean(h2, axis=0, keepdims=True)
    var2 = jnp.mean((h2 - mu2) * (h2 - mu2), axis=0, keepdims=True)
    o_ref[...] = (h2 - mu2) * lax.rsqrt(var2 + BN_EPS) * g2_ref[...] + be2_ref[...]


def _tc_mlp(x, agg, eps, w1t, b1, g1, be1, w2t, b2, g2, be2):
    return pl.pallas_call(
        _tc_mlp_kernel,
        out_shape=jax.ShapeDtypeStruct((N, D), jnp.float32),
    )(x, agg, eps, w1t, b1, g1, be1, w2t, b2, g2, be2)


@jax.jit
def kernel(x, edge_index, eps, W1, b1, g1, be1, W2, b2, g2, be2):
    row4 = edge_index[0].reshape(NC, NS, NBLK, IBLK, CHUNK)
    col4 = edge_index[1].reshape(NC, NS, NBLK, IBLK, CHUNK)
    agg = _sc_agg(x, row4, col4)

    return _tc_mlp(x, agg, eps.reshape(1, 1), W1.T, b1.reshape(1, D),
                   g1.reshape(1, D), be1.reshape(1, D), W2.T,
                   b2.reshape(1, D), g2.reshape(1, D), be2.reshape(1, D))
